# baseline (speedup 1.0000x reference)
import functools

import jax
import jax.numpy as jnp
from jax import lax
from jax.experimental import pallas as pl
from jax.experimental.pallas import tpu as pltpu
from jax.experimental.pallas import tpu_sc as plsc

B = 16384
D = 64
H = 128
NR = 6
NRP = 8
NC = 2
NS = 16
NW = NC * NS
BPW = B // NW
CHUNK = 128
NCHUNK = BPW // CHUNK


def _sc_gather(user_idx, item_idx, user_table, item_table):
    mesh = plsc.VectorSubcoreMesh(core_axis_name="c", subcore_axis_name="s")

    @functools.partial(
        pl.kernel,
        mesh=mesh,
        compiler_params=pltpu.CompilerParams(use_tc_tiling_on_sc=False),
        out_type=(
            jax.ShapeDtypeStruct((B, D), jnp.float32),
            jax.ShapeDtypeStruct((B, D), jnp.float32),
        ),
        scratch_types=[
            pltpu.VMEM((NCHUNK, CHUNK), jnp.int32),
            pltpu.VMEM((NCHUNK, CHUNK), jnp.int32),
            pltpu.VMEM((BPW, D), jnp.float32),
            pltpu.VMEM((BPW, D), jnp.float32),
            pltpu.SemaphoreType.DMA,
        ],
    )
    def gather_kernel(uidx_hbm, iidx_hbm, utab_hbm, itab_hbm,
                      uout_hbm, iout_hbm,
                      uidx_v, iidx_v, urows_v, irows_v, sem):
        wid = lax.axis_index("s") * NC + lax.axis_index("c")
        pltpu.sync_copy(uidx_hbm.at[wid], uidx_v)
        pltpu.sync_copy(iidx_hbm.at[wid], iidx_v)
        copies = []
        for j in range(NCHUNK):
            copies.append(pltpu.async_copy(
                utab_hbm.at[uidx_v.at[j]],
                urows_v.at[pl.ds(j * CHUNK, CHUNK)], sem))
            copies.append(pltpu.async_copy(
                itab_hbm.at[iidx_v.at[j]],
                irows_v.at[pl.ds(j * CHUNK, CHUNK)], sem))
        for c in copies:
            c.wait()
        base = wid * BPW
        pltpu.sync_copy(urows_v, uout_hbm.at[pl.ds(base, BPW)])
        pltpu.sync_copy(irows_v, iout_hbm.at[pl.ds(base, BPW)])

    return gather_kernel(
        user_idx.reshape(NW, NCHUNK, CHUNK),
        item_idx.reshape(NW, NCHUNK, CHUNK),
        user_table, item_table)


BLK = 2048
NB = B // BLK


def _tc_body(u_ref, i_ref, r_ref, rt_ref, w_ref, b_ref, h_ref, re_ref):
    r = r_ref[0, 0, :]
    onehot = (r.reshape(BLK, 1)
              == lax.broadcasted_iota(jnp.int32, (BLK, NRP), 1)
              ).astype(jnp.float32)
    re = jnp.dot(onehot, rt_ref[...], preferred_element_type=jnp.float32)
    cat = jnp.concatenate([u_ref[...], i_ref[...], re], axis=-1)
    h = jnp.tanh(jnp.dot(cat, w_ref[...],
                         preferred_element_type=jnp.float32) + b_ref[...])
    h_ref[...] = h
    re_ref[...] = re


def _tc_encode(user_embed, item_embed, rating, rating_table, W, b):
    rt_pad = jnp.pad(rating_table, ((0, NRP - NR), (0, 0)))
    rating3 = rating.reshape(NB, 1, BLK)
    return pl.pallas_call(
        _tc_body,
        grid=(NB,),
        in_specs=[
            pl.BlockSpec((BLK, D), lambda i: (i, 0)),
            pl.BlockSpec((BLK, D), lambda i: (i, 0)),
            pl.BlockSpec((1, 1, BLK), lambda i: (i, 0, 0)),
            pl.BlockSpec((NRP, D), lambda i: (0, 0)),
            pl.BlockSpec((3 * D, H), lambda i: (0, 0)),
            pl.BlockSpec((1, H), lambda i: (0, 0)),
        ],
        out_specs=[
            pl.BlockSpec((BLK, H), lambda i: (i, 0)),
            pl.BlockSpec((BLK, D), lambda i: (i, 0)),
        ],
        out_shape=[
            jax.ShapeDtypeStruct((B, H), jnp.float32),
            jax.ShapeDtypeStruct((B, D), jnp.float32),
        ],
    )(user_embed, item_embed, rating3, rt_pad, W, b.reshape(1, H))


def kernel(user, item, rating, user_table, item_table, rating_table, W, b):
    user_embed, item_embed = _sc_gather(user, item, user_table, item_table)
    hidden, rating_embed = _tc_encode(
        user_embed, item_embed, rating, rating_table, W, b)
    return (hidden, user_embed, item_embed, rating_embed)
